# baseline (speedup 1.0000x reference)
import jax
import jax.numpy as jnp
from jax.experimental import pallas as pl
from jax.experimental.pallas import tpu as pltpu
from jax.experimental.pallas import tpu_sc as plsc

B = 16384
V = 1000000
T = 128
D = 64
H = 128

_BN = 40960
_QB = _BN // 4
_NBLK = -(-V // _BN)
_QR = _NBLK * _QB
_GATHER_WINDOW = 256
_BM = 2048


def _pack2(hi_bf, lo_bf):
    hi = jax.lax.bitcast_convert_type(hi_bf, jnp.uint16).astype(jnp.uint32)
    lo = jax.lax.bitcast_convert_type(lo_bf, jnp.uint16).astype(jnp.uint32)
    return jax.lax.bitcast_convert_type((hi << 16) | lo, jnp.float32)


def _transpose_body(t_ref, o_ref):
    x = t_ref[...].astype(jnp.bfloat16)
    y = [jnp.transpose(x[:, q * _QB:(q + 1) * _QB], (1, 0))
         for q in range(4)]
    o_ref[:, :D] = _pack2(y[0], y[1])
    o_ref[:, D:] = _pack2(y[2], y[3])


def _tc_transpose(tabT):
    return pl.pallas_call(
        _transpose_body,
        grid=(_NBLK,),
        in_specs=[pl.BlockSpec((D, _BN), lambda i: (0, i))],
        out_specs=pl.BlockSpec((_QB, 2 * D), lambda i: (i, 0)),
        out_shape=jax.ShapeDtypeStruct((_QR, 2 * D), jnp.float32),
        compiler_params=pltpu.CompilerParams(
            dimension_semantics=("arbitrary",)),
    )(tabT)


def _sc_gather(packed, qidx):
    idx2 = qidx.reshape(1, B)
    mesh = plsc.VectorSubcoreMesh(core_axis_name="core",
                                  subcore_axis_name="subcore")

    @pl.kernel(out_type=jax.ShapeDtypeStruct((B, 2 * D), packed.dtype),
               mesh=mesh,
               compiler_params=pltpu.CompilerParams(use_tc_tiling_on_sc=True))
    def gather_kernel(tab_hbm, i_hbm, o_hbm):
        def body(i_vmem, o_vmem):
            pltpu.sync_copy(tab_hbm.at[i_vmem.at[0]], o_vmem)

        pltpu.emit_pipeline(
            body,
            grid=(B // _GATHER_WINDOW,),
            in_specs=[pl.BlockSpec((1, _GATHER_WINDOW),
                                   index_map=lambda i: (0, i))],
            out_specs=[pl.BlockSpec((_GATHER_WINDOW, 2 * D),
                                    index_map=lambda i: (i, 0))],
            core_axis_name=("core", "subcore"),
            dimension_semantics=(pltpu.PARALLEL,),
        )(i_hbm, o_hbm)

    return gather_kernel(packed, idx2)


def _dense_body(pk_hbm, s_ref, tx_ref, wp_ref, bp_ref, wg_ref,
                bg_ref, w1_ref, b1_ref, g_ref, be_ref, w2_ref, b2_ref,
                o_ref, pk_vmem, pk_sem):
    i = pl.program_id(0)
    pltpu.make_async_copy(pk_hbm.at[pl.ds(i * _BM, _BM), :], pk_vmem,
                          pk_sem).start()
    pltpu.make_async_copy(pk_hbm.at[pl.ds(i * _BM, _BM), :], pk_vmem,
                          pk_sem).wait()
    pk = pk_vmem[...]
    s = s_ref[...]
    tx = tx_ref[...]
    f32 = jnp.float32

    w = jnp.where(s >= 2.0, pk[:, D:], pk[:, :D])
    u = jax.lax.bitcast_convert_type(w, jnp.uint32)
    hi = jax.lax.bitcast_convert_type(u & jnp.uint32(0xFFFF0000), f32)
    lo = jax.lax.bitcast_convert_type(u << 16, f32)
    odd = (s == 1.0) | (s == 3.0)
    ids = jnp.where(odd, lo, hi)

    tp = jax.lax.dot_general(tx, wp_ref[...], (((1,), (1,)), ((), ())),
                             preferred_element_type=f32) + bp_ref[...]
    wg = wg_ref[...]
    glog = (jax.lax.dot_general(ids, wg[:, :D], (((1,), (1,)), ((), ())),
                                preferred_element_type=f32)
            + jax.lax.dot_general(tp, wg[:, D:], (((1,), (1,)), ((), ())),
                                  preferred_element_type=f32)
            + bg_ref[...])
    gate = jax.nn.sigmoid(glog)
    fused = gate * ids + (1.0 - gate) * tp

    h = jax.lax.dot_general(fused, w1_ref[...], (((1,), (1,)), ((), ())),
                            preferred_element_type=f32) + b1_ref[...]
    mu = jnp.mean(h, axis=-1, keepdims=True)
    var = jnp.mean((h - mu) ** 2, axis=-1, keepdims=True)
    h = (h - mu) * jax.lax.rsqrt(var + 1e-5) * g_ref[...] + be_ref[...]
    h = jnp.maximum(h, 0.0)

    out = jax.lax.dot_general(h, w2_ref[...], (((1,), (1,)), ((), ())),
                              preferred_element_type=f32) + b2_ref[...]
    nrm = jnp.maximum(jnp.sqrt(jnp.sum(out * out, axis=-1, keepdims=True)),
                      1e-12)
    o_ref[...] = out / nrm


def _tc_dense(packed_g, s_f, text_feat, Wp, bp, Wg, bg, W1, b1,
              ln_g, ln_b, W2, b2):
    full = lambda shape: pl.BlockSpec(shape, lambda i: (0,) * len(shape))
    return pl.pallas_call(
        _dense_body,
        grid=(B // _BM,),
        in_specs=[
            pl.BlockSpec(memory_space=pl.ANY),
            pl.BlockSpec((_BM, 1), lambda i: (i, 0)),
            pl.BlockSpec((_BM, T), lambda i: (i, 0)),
            full((D, T)),
            full((1, D)),
            full((1, 2 * D)),
            full((1, 1)),
            full((H, D)),
            full((1, H)),
            full((1, H)),
            full((1, H)),
            full((H, H)),
            full((1, H)),
        ],
        out_specs=pl.BlockSpec((_BM, H), lambda i: (i, 0)),
        out_shape=jax.ShapeDtypeStruct((B, H), jnp.float32),
        scratch_shapes=[pltpu.VMEM((_BM, 2 * D), jnp.float32),
                        pltpu.SemaphoreType.DMA],
    )(packed_g, s_f, text_feat, Wp, bp.reshape(1, D), Wg,
      bg.reshape(1, 1), W1, b1.reshape(1, H), ln_g.reshape(1, H),
      ln_b.reshape(1, H), W2, b2.reshape(1, H))


@jax.jit
def kernel(text_feat, item_ids, table, Wp, bp, Wg, bg, W1, b1, ln_g, ln_b,
           W2, b2):
    idx = item_ids.astype(jnp.int32)
    packed = _tc_transpose(table.T)
    blk = idx // _BN
    off = idx - blk * _BN
    sub = off // _QB
    qidx = blk * _QB + (off - sub * _QB)
    packed_g = _sc_gather(packed, qidx)
    s_f = sub.astype(jnp.float32).reshape(B, 1)
    return _tc_dense(packed_g, s_f, text_feat, Wp, bp, Wg, bg,
                     W1, b1, ln_g, ln_b, W2, b2)


# speedup vs baseline: 1.0872x; 1.0872x over previous
import jax
import jax.numpy as jnp
from jax.experimental import pallas as pl
from jax.experimental.pallas import tpu as pltpu
from jax.experimental.pallas import tpu_sc as plsc

B = 16384
V = 1000000
T = 128
D = 64
H = 128

_BN = 40960
_QB = _BN // 4
_NBLK = -(-V // _BN)
_QR = _NBLK * _QB
_GATHER_WINDOW = 256
_BM = 2048


def _pack2(hi_bf, lo_bf):
    hi = jax.lax.bitcast_convert_type(hi_bf, jnp.uint16).astype(jnp.uint32)
    lo = jax.lax.bitcast_convert_type(lo_bf, jnp.uint16).astype(jnp.uint32)
    return jax.lax.bitcast_convert_type((hi << 16) | lo, jnp.float32)


def _transpose_body(t_ref, o_ref):
    x = t_ref[...].astype(jnp.bfloat16)
    y = [jnp.transpose(x[:, q * _QB:(q + 1) * _QB], (1, 0))
         for q in range(4)]
    o_ref[:, :D] = _pack2(y[0], y[1])
    o_ref[:, D:] = _pack2(y[2], y[3])


def _tc_transpose(tabT):
    return pl.pallas_call(
        _transpose_body,
        grid=(_NBLK,),
        in_specs=[pl.BlockSpec((D, _BN), lambda i: (0, i))],
        out_specs=pl.BlockSpec((_QB, 2 * D), lambda i: (i, 0)),
        out_shape=jax.ShapeDtypeStruct((_QR, 2 * D), jnp.float32),
        compiler_params=pltpu.CompilerParams(
            dimension_semantics=("arbitrary",)),
    )(tabT)


def _sc_gather(packed, qidx):
    idx2 = qidx.reshape(1, B)
    mesh = plsc.VectorSubcoreMesh(core_axis_name="core",
                                  subcore_axis_name="subcore")

    @pl.kernel(out_type=jax.ShapeDtypeStruct((B, 2 * D), packed.dtype),
               mesh=mesh,
               compiler_params=pltpu.CompilerParams(use_tc_tiling_on_sc=True))
    def gather_kernel(tab_hbm, i_hbm, o_hbm):
        def body(i_vmem, o_vmem):
            pltpu.sync_copy(tab_hbm.at[i_vmem.at[0]], o_vmem)

        pltpu.emit_pipeline(
            body,
            grid=(B // _GATHER_WINDOW,),
            in_specs=[pl.BlockSpec((1, _GATHER_WINDOW),
                                   index_map=lambda i: (0, i))],
            out_specs=[pl.BlockSpec((_GATHER_WINDOW, 2 * D),
                                    index_map=lambda i: (i, 0))],
            core_axis_name=("core", "subcore"),
            dimension_semantics=(pltpu.PARALLEL,),
        )(i_hbm, o_hbm)

    return gather_kernel(packed, idx2)


def _dense_body(pk_ref, s_ref, tx_ref, wp_ref, bp_ref, wg_ref,
                bg_ref, w1_ref, b1_ref, g_ref, be_ref, w2_ref, b2_ref,
                o_ref):
    pk = pk_ref[...]
    s = s_ref[...]
    tx = tx_ref[...]
    f32 = jnp.float32

    w = jnp.where(s >= 2.0, pk[:, D:], pk[:, :D])
    u = jax.lax.bitcast_convert_type(w, jnp.uint32)
    hi = jax.lax.bitcast_convert_type(u & jnp.uint32(0xFFFF0000), f32)
    lo = jax.lax.bitcast_convert_type(u << 16, f32)
    odd = (s == 1.0) | (s == 3.0)
    ids = jnp.where(odd, lo, hi)

    tp = jax.lax.dot_general(tx, wp_ref[...], (((1,), (1,)), ((), ())),
                             preferred_element_type=f32) + bp_ref[...]
    wg = wg_ref[...]
    glog = (jax.lax.dot_general(ids, wg[:, :D], (((1,), (1,)), ((), ())),
                                preferred_element_type=f32)
            + jax.lax.dot_general(tp, wg[:, D:], (((1,), (1,)), ((), ())),
                                  preferred_element_type=f32)
            + bg_ref[...])
    gate = jax.nn.sigmoid(glog)
    fused = gate * ids + (1.0 - gate) * tp

    h = jax.lax.dot_general(fused, w1_ref[...], (((1,), (1,)), ((), ())),
                            preferred_element_type=f32) + b1_ref[...]
    mu = jnp.mean(h, axis=-1, keepdims=True)
    var = jnp.mean((h - mu) ** 2, axis=-1, keepdims=True)
    h = (h - mu) * jax.lax.rsqrt(var + 1e-5) * g_ref[...] + be_ref[...]
    h = jnp.maximum(h, 0.0)

    out = jax.lax.dot_general(h, w2_ref[...], (((1,), (1,)), ((), ())),
                              preferred_element_type=f32) + b2_ref[...]
    nrm = jnp.maximum(jnp.sqrt(jnp.sum(out * out, axis=-1, keepdims=True)),
                      1e-12)
    o_ref[...] = out / nrm


def _tc_dense(packed_g, s_f, text_feat, Wp, bp, Wg, bg, W1, b1,
              ln_g, ln_b, W2, b2):
    full = lambda shape: pl.BlockSpec(shape, lambda i: (0,) * len(shape))
    return pl.pallas_call(
        _dense_body,
        grid=(B // _BM,),
        in_specs=[
            pl.BlockSpec((_BM, 2 * D), lambda i: (i, 0)),
            pl.BlockSpec((_BM, 1), lambda i: (i, 0)),
            pl.BlockSpec((_BM, T), lambda i: (i, 0)),
            full((D, T)),
            full((1, D)),
            full((1, 2 * D)),
            full((1, 1)),
            full((H, D)),
            full((1, H)),
            full((1, H)),
            full((1, H)),
            full((H, H)),
            full((1, H)),
        ],
        out_specs=pl.BlockSpec((_BM, H), lambda i: (i, 0)),
        out_shape=jax.ShapeDtypeStruct((B, H), jnp.float32),
    )(packed_g, s_f, text_feat, Wp, bp.reshape(1, D), Wg,
      bg.reshape(1, 1), W1, b1.reshape(1, H), ln_g.reshape(1, H),
      ln_b.reshape(1, H), W2, b2.reshape(1, H))


@jax.jit
def kernel(text_feat, item_ids, table, Wp, bp, Wg, bg, W1, b1, ln_g, ln_b,
           W2, b2):
    idx = item_ids.astype(jnp.int32)
    packed = _tc_transpose(table.T)
    blk = idx // _BN
    off = idx - blk * _BN
    sub = off // _QB
    qidx = blk * _QB + (off - sub * _QB)
    packed_g = _sc_gather(packed, qidx)
    s_f = sub.astype(jnp.float32).reshape(B, 1)
    return _tc_dense(packed_g, s_f, text_feat, Wp, bp, Wg, bg,
                     W1, b1, ln_g, ln_b, W2, b2)


# speedup vs baseline: 1.0993x; 1.0111x over previous
import jax
import jax.numpy as jnp
from jax.experimental import pallas as pl
from jax.experimental.pallas import tpu as pltpu
from jax.experimental.pallas import tpu_sc as plsc

B = 16384
V = 1000000
T = 128
D = 64
H = 128

_BN = 51200
_QB = _BN // 4
_NBLK = -(-V // _BN)
_QR = _NBLK * _QB
_GATHER_WINDOW = 256
_BM = 2048


def _pack2(hi_bf, lo_bf):
    hi = jax.lax.bitcast_convert_type(hi_bf, jnp.uint16).astype(jnp.uint32)
    lo = jax.lax.bitcast_convert_type(lo_bf, jnp.uint16).astype(jnp.uint32)
    return jax.lax.bitcast_convert_type((hi << 16) | lo, jnp.float32)


def _transpose_body(t_ref, o_ref):
    x = t_ref[...].astype(jnp.bfloat16)
    y = [jnp.transpose(x[:, q * _QB:(q + 1) * _QB], (1, 0))
         for q in range(4)]
    o_ref[:, :D] = _pack2(y[0], y[1])
    o_ref[:, D:] = _pack2(y[2], y[3])


def _tc_transpose(tabT):
    return pl.pallas_call(
        _transpose_body,
        grid=(_NBLK,),
        in_specs=[pl.BlockSpec((D, _BN), lambda i: (0, i))],
        out_specs=pl.BlockSpec((_QB, 2 * D), lambda i: (i, 0)),
        out_shape=jax.ShapeDtypeStruct((_QR, 2 * D), jnp.float32),
        compiler_params=pltpu.CompilerParams(
            dimension_semantics=("arbitrary",),
            vmem_limit_bytes=100 * 1024 * 1024),
    )(tabT)


def _sc_gather(packed, qidx):
    idx2 = qidx.reshape(1, B)
    mesh = plsc.VectorSubcoreMesh(core_axis_name="core",
                                  subcore_axis_name="subcore")

    @pl.kernel(out_type=jax.ShapeDtypeStruct((B, 2 * D), packed.dtype),
               mesh=mesh,
               compiler_params=pltpu.CompilerParams(use_tc_tiling_on_sc=True))
    def gather_kernel(tab_hbm, i_hbm, o_hbm):
        def body(i_vmem, o_vmem):
            pltpu.sync_copy(tab_hbm.at[i_vmem.at[0]], o_vmem)

        pltpu.emit_pipeline(
            body,
            grid=(B // _GATHER_WINDOW,),
            in_specs=[pl.BlockSpec((1, _GATHER_WINDOW),
                                   index_map=lambda i: (0, i))],
            out_specs=[pl.BlockSpec((_GATHER_WINDOW, 2 * D),
                                    index_map=lambda i: (i, 0))],
            core_axis_name=("core", "subcore"),
            dimension_semantics=(pltpu.PARALLEL,),
        )(i_hbm, o_hbm)

    return gather_kernel(packed, idx2)


def _dense_body(pk_ref, s_ref, tx_ref, wp_ref, bp_ref, wg_ref,
                bg_ref, w1_ref, b1_ref, g_ref, be_ref, w2_ref, b2_ref,
                o_ref):
    pk = pk_ref[...]
    s = s_ref[...]
    tx = tx_ref[...]
    f32 = jnp.float32

    w = jnp.where(s >= 2.0, pk[:, D:], pk[:, :D])
    u = jax.lax.bitcast_convert_type(w, jnp.uint32)
    hi = jax.lax.bitcast_convert_type(u & jnp.uint32(0xFFFF0000), f32)
    lo = jax.lax.bitcast_convert_type(u << 16, f32)
    odd = (s == 1.0) | (s == 3.0)
    ids = jnp.where(odd, lo, hi)

    tp = jax.lax.dot_general(tx, wp_ref[...], (((1,), (1,)), ((), ())),
                             preferred_element_type=f32) + bp_ref[...]
    wg = wg_ref[...]
    glog = (jax.lax.dot_general(ids, wg[:, :D], (((1,), (1,)), ((), ())),
                                preferred_element_type=f32)
            + jax.lax.dot_general(tp, wg[:, D:], (((1,), (1,)), ((), ())),
                                  preferred_element_type=f32)
            + bg_ref[...])
    gate = jax.nn.sigmoid(glog)
    fused = gate * ids + (1.0 - gate) * tp

    h = jax.lax.dot_general(fused, w1_ref[...], (((1,), (1,)), ((), ())),
                            preferred_element_type=f32) + b1_ref[...]
    mu = jnp.mean(h, axis=-1, keepdims=True)
    var = jnp.mean((h - mu) ** 2, axis=-1, keepdims=True)
    h = (h - mu) * jax.lax.rsqrt(var + 1e-5) * g_ref[...] + be_ref[...]
    h = jnp.maximum(h, 0.0)

    out = jax.lax.dot_general(h, w2_ref[...], (((1,), (1,)), ((), ())),
                              preferred_element_type=f32) + b2_ref[...]
    nrm = jnp.maximum(jnp.sqrt(jnp.sum(out * out, axis=-1, keepdims=True)),
                      1e-12)
    o_ref[...] = out / nrm


def _tc_dense(packed_g, s_f, text_feat, Wp, bp, Wg, bg, W1, b1,
              ln_g, ln_b, W2, b2):
    full = lambda shape: pl.BlockSpec(shape, lambda i: (0,) * len(shape))
    return pl.pallas_call(
        _dense_body,
        grid=(B // _BM,),
        in_specs=[
            pl.BlockSpec((_BM, 2 * D), lambda i: (i, 0)),
            pl.BlockSpec((_BM, 1), lambda i: (i, 0)),
            pl.BlockSpec((_BM, T), lambda i: (i, 0)),
            full((D, T)),
            full((1, D)),
            full((1, 2 * D)),
            full((1, 1)),
            full((H, D)),
            full((1, H)),
            full((1, H)),
            full((1, H)),
            full((H, H)),
            full((1, H)),
        ],
        out_specs=pl.BlockSpec((_BM, H), lambda i: (i, 0)),
        out_shape=jax.ShapeDtypeStruct((B, H), jnp.float32),
    )(packed_g, s_f, text_feat, Wp, bp.reshape(1, D), Wg,
      bg.reshape(1, 1), W1, b1.reshape(1, H), ln_g.reshape(1, H),
      ln_b.reshape(1, H), W2, b2.reshape(1, H))


@jax.jit
def kernel(text_feat, item_ids, table, Wp, bp, Wg, bg, W1, b1, ln_g, ln_b,
           W2, b2):
    idx = item_ids.astype(jnp.int32)
    packed = _tc_transpose(table.T)
    blk = idx // _BN
    off = idx - blk * _BN
    sub = off // _QB
    qidx = blk * _QB + (off - sub * _QB)
    packed_g = _sc_gather(packed, qidx)
    s_f = sub.astype(jnp.float32).reshape(B, 1)
    return _tc_dense(packed_g, s_f, text_feat, Wp, bp, Wg, bg,
                     W1, b1, ln_g, ln_b, W2, b2)
